# baseline (speedup 1.0000x reference)
import functools

import jax
import jax.numpy as jnp
from jax import lax
from jax.experimental import pallas as pl
from jax.experimental.pallas import tpu as pltpu
from jax.experimental.pallas import tpu_sc as plsc

VOCAB = 1000
VPAD = 1024
NTC = VPAD // 128
D_MODEL = 64
BATCH = 1024
SEQ = 50
TPAD = 56

_info = plsc.get_sparse_core_info()
NC, NS = _info.num_cores, _info.num_subcores
NW = NC * NS
B_PER_W = BATCH // NW
IDX_PER_W = B_PER_W * NTC * TPAD


def _mm_body(a_ref, b_ref, o_ref):
    o_ref[...] = lax.dot_general(
        a_ref[...], b_ref[...],
        (((1,), (1,)), ((), ())),
        preferred_element_type=jnp.float32,
    )


def _token_logit_table(embed_table, head_w_pad):
    return pl.pallas_call(
        _mm_body,
        out_shape=jax.ShapeDtypeStruct((VOCAB, VPAD), jnp.float32),
    )(embed_table, head_w_pad)


_mesh = plsc.VectorSubcoreMesh(core_axis_name="c", subcore_axis_name="s")


@functools.partial(
    pl.kernel,
    mesh=_mesh,
    out_type=jax.ShapeDtypeStruct((BATCH, SEQ, VOCAB), jnp.float32),
    scratch_types=[
        pltpu.VMEM((NTC * TPAD,), jnp.int32),
        pltpu.VMEM((NTC * TPAD,), jnp.int32),
        pltpu.VMEM((SEQ, VOCAB), jnp.float32),
        pltpu.VMEM((SEQ, VOCAB), jnp.float32),
        pltpu.VMEM((SEQ, 128), jnp.float32),
        pltpu.VMEM((SEQ, 128), jnp.float32),
        pltpu.SemaphoreType.DMA,
        pltpu.SemaphoreType.DMA,
    ],
)
def _gather_rows(m8_hbm, idx_hbm, out_hbm, idx0, idx1, buf0, buf1, tl0, tl1, sem0, sem1):
    wid = lax.axis_index("s") * NC + lax.axis_index("c")

    def copies(idx_v, buf, tl, sem):
        cs = [
            pltpu.make_async_copy(
                m8_hbm.at[idx_v.at[pl.ds(tc * TPAD, SEQ)]],
                buf.at[:, pl.ds(128 * tc, 128)],
                sem,
            )
            for tc in range(NTC - 1)
        ]
        cs.append(
            pltpu.make_async_copy(
                m8_hbm.at[idx_v.at[pl.ds((NTC - 1) * TPAD, SEQ)]],
                tl,
                sem,
            )
        )
        return cs

    def start(j, idx_v, buf, tl, sem):
        pltpu.sync_copy(
            idx_hbm.at[pl.ds((wid * B_PER_W + j) * NTC * TPAD, NTC * TPAD)], idx_v
        )
        for c in copies(idx_v, buf, tl, sem):
            c.start()

    _TAIL_OFF = (NTC - 1) * 128
    _TAIL = VOCAB - _TAIL_OFF

    def splice_tail(buf, tl):
        def row(t, carry):
            for k in range(_TAIL // 16):
                buf[t, pl.ds(_TAIL_OFF + 16 * k, 16)] = tl[t, pl.ds(16 * k, 16)]
            buf[t, pl.ds(VOCAB - 16, 16)] = tl[t, pl.ds(_TAIL - 16, 16)]
            return carry

        lax.fori_loop(0, SEQ, row, 0)

    def finish(j, idx_v, buf, tl, sem):
        for c in copies(idx_v, buf, tl, sem):
            c.wait()
        splice_tail(buf, tl)
        pltpu.sync_copy(buf, out_hbm.at[wid * B_PER_W + j])

    start(0, idx0, buf0, tl0, sem0)

    def body(g, carry):
        j0 = 2 * g
        start(j0 + 1, idx1, buf1, tl1, sem1)
        finish(j0, idx0, buf0, tl0, sem0)

        @pl.when(j0 + 2 < B_PER_W)
        def _():
            start(j0 + 2, idx0, buf0, tl0, sem0)

        finish(j0 + 1, idx1, buf1, tl1, sem1)
        return carry

    lax.fori_loop(0, B_PER_W // 2, body, 0)


def kernel(input_ids, embed_table, head_w):
    head_pad = jnp.pad(head_w, ((0, VPAD - VOCAB), (0, 0)))
    m = _token_logit_table(embed_table, head_pad)
    m8 = m.reshape(VOCAB * NTC, 128)
    ids = input_ids.astype(jnp.int32)
    idx_all = (NTC * ids)[:, None, :] + jnp.arange(NTC, dtype=jnp.int32)[None, :, None]
    idx_all = jnp.pad(idx_all, ((0, 0), (0, 0), (0, TPAD - SEQ)))
    return _gather_rows(m8, idx_all.reshape(-1))
